# baseline (speedup 1.0000x reference)
import jax
import jax.numpy as jnp
from jax import lax
from jax.experimental import pallas as pl
from jax.experimental.pallas import tpu as pltpu
from jax.experimental.pallas import tpu_sc as plsc

B = 4096
L = 200
WDIM = 64
PDIM = 16
ODIM = WDIM + 2 * PDIM
VOCAB1 = 100001
N = B * L
NC = 2
NS = 16
NW = NC * NS
ROWS = B // NW
PTAB = 200 * PDIM
GROUP_STARTS = tuple(range(0, 192, 16)) + (184,)


def _emb_body(word_hbm, pos1_hbm, pos2_hbm, wtab_hbm, p1_hbm, p2_hbm,
              out_hbm,
              wiA, p1iA, p2iA, wiB, p1iB, p2iB, wrowsA, wrowsB,
              obufA, obufB, p1t, p2t,
              semiA, semiB, semgA, semgB, semoA, semoB, semt):
    c = lax.axis_index("c")
    s = lax.axis_index("s")
    wid = s * NC + c
    row0 = wid * ROWS

    bufs = (
        (wiA, p1iA, p2iA, wrowsA, obufA, semiA, semgA, semoA),
        (wiB, p1iB, p2iB, wrowsB, obufB, semiB, semgB, semoB),
    )

    def idx_copies(i, par):
        wi, p1i, p2i, _, _, semi, _, _ = bufs[par]
        t0 = (row0 + i) * L
        return [
            pltpu.make_async_copy(word_hbm.at[pl.ds(t0, L)], wi, semi),
            pltpu.make_async_copy(pos1_hbm.at[pl.ds(t0, L)], p1i, semi),
            pltpu.make_async_copy(pos2_hbm.at[pl.ds(t0, L)], p2i, semi),
        ]

    def gather_copies(par):
        wi, _, _, wrows, _, _, semg, _ = bufs[par]
        return [
            pltpu.make_async_copy(wtab_hbm.at[wi.at[pl.ds(0, 104)]],
                                  wrows.at[pl.ds(0, 104)], semg),
            pltpu.make_async_copy(wtab_hbm.at[wi.at[pl.ds(104, 96)]],
                                  wrows.at[pl.ds(104, 96)], semg),
        ]

    def out_copies(i, par):
        _, _, _, _, obuf, _, _, semo = bufs[par]
        return [pltpu.make_async_copy(obuf, out_hbm.at[row0 + i], semo)]

    def fire(copies):
        for cp in copies:
            cp.start()

    def drain(copies):
        for cp in copies:
            cp.wait()

    def merge(par):
        _, p1i, p2i, wrows, obuf, _, _, _ = bufs[par]

        def copy_word(u, carry):
            t = 2 * u
            for dt in range(2):
                for j in range(WDIM // 16):
                    obuf[t + dt, pl.ds(16 * j, 16)] = (
                        wrows[t + dt, pl.ds(16 * j, 16)])
            return carry

        lax.fori_loop(0, L // 2, copy_word, 0)

        lanes = lax.iota(jnp.int32, 16)
        for t0 in GROUP_STARTS:
            tok = lanes + t0
            a1 = p1i[pl.ds(t0, 16)] * PDIM
            a2 = p2i[pl.ds(t0, 16)] * PDIM
            for cc in range(PDIM):
                col1 = jnp.full((16,), WDIM + cc, jnp.int32)
                v1 = plsc.load_gather(p1t, [a1 + cc])
                plsc.store_scatter(obuf, [tok, col1], v1)
                col2 = jnp.full((16,), WDIM + PDIM + cc, jnp.int32)
                v2 = plsc.load_gather(p2t, [a2 + cc])
                plsc.store_scatter(obuf, [tok, col2], v2)

    pltpu.make_async_copy(p1_hbm, p1t, semt).start()
    pltpu.make_async_copy(p2_hbm, p2t, semt).start()
    pltpu.make_async_copy(p1_hbm, p1t, semt).wait()
    pltpu.make_async_copy(p2_hbm, p2t, semt).wait()

    fire(idx_copies(0, 0))
    fire(idx_copies(1, 1))

    def seq(i, par, k, first):
        drain(idx_copies(i, par))

        @pl.when(k >= 1)
        def _():
            drain(out_copies(i - 2, par))

        fire(gather_copies(par))

        def tail():
            opar = 1 - par
            drain(gather_copies(opar))
            merge(opar)
            fire(out_copies(i - 1, opar))
            if first:
                fire(idx_copies(i + 1, opar))
            else:
                pl.when(k < ROWS // 2 - 1)(
                    lambda: fire(idx_copies(i + 1, opar)))

        if first:
            pl.when(k >= 1)(tail)
        else:
            tail()

    def body(k, carry):
        seq(2 * k, 0, k, True)
        seq(2 * k + 1, 1, k, False)
        return carry

    lax.fori_loop(0, ROWS // 2, body, 0)

    drain(gather_copies(1))
    merge(1)
    fire(out_copies(ROWS - 1, 1))
    drain(out_copies(ROWS - 2, 0))
    drain(out_copies(ROWS - 1, 1))


@jax.jit
def _run(word_flat, pos1_flat, pos2_flat, wtabp, p1_flat, p2_flat):
    mesh = plsc.VectorSubcoreMesh(core_axis_name="c", subcore_axis_name="s")
    f = pl.kernel(
        _emb_body,
        mesh=mesh,
        compiler_params=pltpu.CompilerParams(needs_layout_passes=False),
        out_type=jax.ShapeDtypeStruct((B, L, ODIM), jnp.float32),
        scratch_types=[
            pltpu.VMEM((L,), jnp.int32),
            pltpu.VMEM((L,), jnp.int32),
            pltpu.VMEM((L,), jnp.int32),
            pltpu.VMEM((L,), jnp.int32),
            pltpu.VMEM((L,), jnp.int32),
            pltpu.VMEM((L,), jnp.int32),
            pltpu.VMEM((L, 2 * WDIM), jnp.float32),
            pltpu.VMEM((L, 2 * WDIM), jnp.float32),
            pltpu.VMEM((L, ODIM), jnp.float32),
            pltpu.VMEM((L, ODIM), jnp.float32),
            pltpu.VMEM((PTAB,), jnp.float32),
            pltpu.VMEM((PTAB,), jnp.float32),
            pltpu.SemaphoreType.DMA,
            pltpu.SemaphoreType.DMA,
            pltpu.SemaphoreType.DMA,
            pltpu.SemaphoreType.DMA,
            pltpu.SemaphoreType.DMA,
            pltpu.SemaphoreType.DMA,
            pltpu.SemaphoreType.DMA,
        ],
    )
    return f(word_flat, pos1_flat, pos2_flat, wtabp, p1_flat, p2_flat)


def kernel(word, pos1, pos2, word_table, pos1_table, pos2_table):
    word_flat = word.reshape(N).astype(jnp.int32)
    pos1_flat = pos1.reshape(N).astype(jnp.int32)
    pos2_flat = pos2.reshape(N).astype(jnp.int32)
    wtabp = jnp.pad(word_table, ((0, 0), (0, 2 * WDIM - WDIM)))
    p1_flat = pos1_table.at[0].set(0.0).reshape(PTAB)
    p2_flat = pos2_table.at[0].set(0.0).reshape(PTAB)
    return _run(word_flat, pos1_flat, pos2_flat, wtabp, p1_flat, p2_flat)


# speedup vs baseline: 1.2150x; 1.2150x over previous
import jax
import jax.numpy as jnp
from jax import lax
from jax.experimental import pallas as pl
from jax.experimental.pallas import tpu as pltpu
from jax.experimental.pallas import tpu_sc as plsc

B = 4096
L = 200
WDIM = 64
PDIM = 16
ODIM = WDIM + 2 * PDIM
N = B * L
NC = 2
NS = 16
NW = NC * NS
CHUNK = N // NW
T = 512
STEPS = CHUNK // T


def _emb_body(word_hbm, pos1_hbm, pos2_hbm, wtab_hbm, p1tab_hbm, p2tab_hbm,
              out_hbm, widx_v, p1idx_v, p2idx_v, wrows_v, p1rows_v, p2rows_v,
              sem_i, sem_g, sem_o):
    c = lax.axis_index("c")
    s = lax.axis_index("s")
    wid = s * NC + c
    base0 = wid * CHUNK

    def idx_copies(i, p):
        base = base0 + i * T
        return [
            pltpu.make_async_copy(word_hbm.at[pl.ds(base, T)],
                                  widx_v.at[p], sem_i.at[p]),
            pltpu.make_async_copy(pos1_hbm.at[pl.ds(base, T)],
                                  p1idx_v.at[p], sem_i.at[p]),
            pltpu.make_async_copy(pos2_hbm.at[pl.ds(base, T)],
                                  p2idx_v.at[p], sem_i.at[p]),
        ]

    def gather_copies(p):
        return [
            pltpu.make_async_copy(wtab_hbm.at[widx_v.at[p]],
                                  wrows_v.at[p], sem_g.at[p]),
            pltpu.make_async_copy(p1tab_hbm.at[p1idx_v.at[p]],
                                  p1rows_v.at[p], sem_g.at[p]),
            pltpu.make_async_copy(p2tab_hbm.at[p2idx_v.at[p]],
                                  p2rows_v.at[p], sem_g.at[p]),
        ]

    def out_copies(i, p):
        base = base0 + i * T
        return [
            pltpu.make_async_copy(
                wrows_v.at[p],
                out_hbm.at[pl.ds(base, T), pl.ds(0, WDIM)], sem_o.at[p]),
            pltpu.make_async_copy(
                p1rows_v.at[p],
                out_hbm.at[pl.ds(base, T), pl.ds(WDIM, PDIM)], sem_o.at[p]),
            pltpu.make_async_copy(
                p2rows_v.at[p],
                out_hbm.at[pl.ds(base, T), pl.ds(WDIM + PDIM, PDIM)],
                sem_o.at[p]),
        ]

    def fire(copies):
        for cp in copies:
            cp.start()

    def drain(copies):
        for cp in copies:
            cp.wait()

    fire(idx_copies(0, 0))
    fire(idx_copies(1, 1))
    drain(idx_copies(0, 0))
    fire(gather_copies(0))

    def step(i, carry):
        p = i & 1
        q = 1 - p
        drain(gather_copies(q))
        fire(out_copies(i - 1, q))

        @pl.when(i + 1 < STEPS)
        def _():
            fire(idx_copies(i + 1, q))

        drain(idx_copies(i, p))

        @pl.when(i >= 2)
        def _():
            drain(out_copies(i - 2, p))

        fire(gather_copies(p))
        return carry

    lax.fori_loop(1, STEPS, step, 0)

    qe = (STEPS - 1) & 1
    drain(gather_copies(qe))
    fire(out_copies(STEPS - 1, qe))
    drain(out_copies(STEPS - 2, 1 - qe))
    drain(out_copies(STEPS - 1, qe))


@jax.jit
def _run(word_flat, pos1_flat, pos2_flat, word_table, p1_tab, p2_tab):
    mesh = plsc.VectorSubcoreMesh(core_axis_name="c", subcore_axis_name="s")
    f = pl.kernel(
        _emb_body,
        mesh=mesh,
        compiler_params=pltpu.CompilerParams(use_tc_tiling_on_sc=False),
        out_type=jax.ShapeDtypeStruct((N, ODIM), jnp.float32),
        scratch_types=[
            pltpu.VMEM((2, T), jnp.int32),
            pltpu.VMEM((2, T), jnp.int32),
            pltpu.VMEM((2, T), jnp.int32),
            pltpu.VMEM((2, T, WDIM), jnp.float32),
            pltpu.VMEM((2, T, PDIM), jnp.float32),
            pltpu.VMEM((2, T, PDIM), jnp.float32),
            pltpu.SemaphoreType.DMA((2,)),
            pltpu.SemaphoreType.DMA((2,)),
            pltpu.SemaphoreType.DMA((2,)),
        ],
    )
    return f(word_flat, pos1_flat, pos2_flat, word_table, p1_tab, p2_tab)


def kernel(word, pos1, pos2, word_table, pos1_table, pos2_table):
    word_flat = word.reshape(N).astype(jnp.int32)
    pos1_flat = pos1.reshape(N).astype(jnp.int32)
    pos2_flat = pos2.reshape(N).astype(jnp.int32)
    p1_tab = pos1_table.at[0].set(0.0)
    p2_tab = pos2_table.at[0].set(0.0)
    out = _run(word_flat, pos1_flat, pos2_flat, word_table, p1_tab, p2_tab)
    return out.reshape(B, L, ODIM)
